# baseline (speedup 1.0000x reference)
import functools

import jax
import jax.numpy as jnp
from jax import lax
from jax.experimental import pallas as pl
from jax.experimental.pallas import tpu as pltpu
from jax.experimental.pallas import tpu_sc as plsc

N = 10000
E = 320000
D = 128
H = 64
C = 128
G = 128

NC = 2
NS = 16
L = 16
FPT = H // NS
CHUNK = 8192
NCH = -(-E // CHUNK)
EPAD = NCH * CHUNK
PAD = EPAD - E
GROUPS = CHUNK // L
RPT = 632
N_PAD = NS * RPT
PT4 = N_PAD * FPT


@functools.lru_cache(maxsize=1)
def _make_segsum():
    mesh = plsc.VectorSubcoreMesh(core_axis_name="c", subcore_axis_name="s",
                                  num_cores=NC, num_subcores=NS)

    @functools.partial(
        pl.kernel,
        out_type=jax.ShapeDtypeStruct((NC * NS, PT4), jnp.float32),
        mesh=mesh,
        compiler_params=pltpu.CompilerParams(use_tc_tiling_on_sc=False,
                                             needs_layout_passes=False),
        scratch_types=[
            pltpu.VMEM((CHUNK,), jnp.int32),
            pltpu.VMEM((CHUNK,), jnp.int32),
            pltpu.VMEM((CHUNK,), jnp.int32),
            pltpu.VMEM((CHUNK,), jnp.int32),
            pltpu.VMEM((PT4,), jnp.float32),
            pltpu.VMEM((PT4,), jnp.float32),
            pltpu.SemaphoreType.DMA,
            pltpu.SemaphoreType.DMA,
        ],
    )
    def seg(y4_hbm, srcq_hbm, dstq_hbm, z_hbm, out_hbm,
            src0, src1, dst0, dst1, y_t, acc, gsem, dsem):
        c = lax.axis_index("c")
        s = lax.axis_index("s")
        w = c * NS + s
        pltpu.sync_copy(y4_hbm.at[w], y_t)
        pltpu.sync_copy(z_hbm.at[pl.ds(s * PT4, PT4)], acc)
        base = c * NCH
        pltpu.async_copy(srcq_hbm.at[base], src0, gsem)
        pltpu.async_copy(dstq_hbm.at[base], dst0, dsem)

        def run_chunk(src_b, dst_b):
            def group(g, carry):
                s16 = src_b[pl.ds(g * L, L)]
                d16 = dst_b[pl.ds(g * L, L)]
                s4 = s16 * FPT
                d4 = d16 * FPT
                for f in range(FPT):
                    si = s4 if f == 0 else s4 + f
                    di = d4 if f == 0 else d4 + f
                    v = plsc.load_gather(y_t, [si])
                    plsc.addupdate_scatter(acc, [di], v)
                return carry
            lax.fori_loop(0, GROUPS, group, 0)

        def chunk_pair(i, carry):
            ch0 = 2 * i
            ch1 = ch0 + 1
            pltpu.make_async_copy(srcq_hbm.at[base], src0, gsem).wait()
            pltpu.make_async_copy(dstq_hbm.at[base], dst0, dsem).wait()
            pltpu.async_copy(srcq_hbm.at[base + ch1], src1, gsem)
            pltpu.async_copy(dstq_hbm.at[base + ch1], dst1, dsem)
            run_chunk(src0, dst0)
            pltpu.make_async_copy(srcq_hbm.at[base], src1, gsem).wait()
            pltpu.make_async_copy(dstq_hbm.at[base], dst1, dsem).wait()
            ch2 = lax.min(ch1 + 1, NCH - 1)
            pltpu.async_copy(srcq_hbm.at[base + ch2], src0, gsem)
            pltpu.async_copy(dstq_hbm.at[base + ch2], dst0, dsem)
            run_chunk(src1, dst1)
            return carry

        lax.fori_loop(0, NCH // 2, chunk_pair, 0)
        pltpu.make_async_copy(srcq_hbm.at[base], src0, gsem).wait()
        pltpu.make_async_copy(dstq_hbm.at[base], dst0, dsem).wait()
        pltpu.sync_copy(acc, out_hbm.at[w])

    return seg


def _pre_body(x1, x2, w1, y):
    w = w1[...]
    z = jnp.zeros((N_PAD - N, H), jnp.float32)
    for br, x in enumerate((x1, x2)):
        y[br, :N] = jnp.dot(x[...], w, preferred_element_type=jnp.float32)
        y[br, N:] = z


def _bn(h, g, b):
    m = jnp.mean(h, axis=0, keepdims=True)
    v = jnp.mean((h - m) ** 2, axis=0, keepdims=True)
    return (h - m) * lax.rsqrt(v + 1e-5) * g + b


def _post_body(y, a, b1, bng, bnb, w2, b2, bg, bb, wn, h_out, yn_out):
    for br in range(2):
        h = y[br, :N] + a[br, :N] + b1[...]
        h = jnp.maximum(_bn(h, bng[...], bnb[...]), 0.0)
        h = jnp.dot(h, w2[...], preferred_element_type=jnp.float32) + b2[...]
        h = jnp.maximum(_bn(h, bg[...], bb[...]), 0.0)
        h_out[br] = h
        if yn_out is not None:
            yn_out[br, :N] = jnp.dot(h, wn[...],
                                     preferred_element_type=jnp.float32)
            yn_out[br, N:] = jnp.zeros((N_PAD - N, H), jnp.float32)


def _post_last_body(y, a, b1, bng, bnb, w2, b2, bg, bb, h_out):
    _post_body(y, a, b1, bng, bnb, w2, b2, bg, bb, None, h_out, None)


_CONTRACT0 = (((0,), (0,)), ((), ()))


def _pool_body(x1, x2, h1, h2, bt1, bt2, l0w, l1w, l2w,
               l0b, l1b, l2b, dw, db, dist, o1, o2):
    xs = (x1, x2)
    bts = (bt1, bt2)
    outs = (o1, o2)
    lb = l0b[...] + l1b[...] + l2b[...]
    dcol = db[...]
    for br in range(2):
        oh = (bts[br][...] == lax.broadcasted_iota(jnp.int32, (N, G), 1)
              ).astype(jnp.float32)
        p0 = lax.dot_general(oh, xs[br][...], _CONTRACT0,
                             preferred_element_type=jnp.float32)
        p1 = lax.dot_general(oh, h1[br], _CONTRACT0,
                             preferred_element_type=jnp.float32)
        p2 = lax.dot_general(oh, h2[br], _CONTRACT0,
                             preferred_element_type=jnp.float32)
        outs[br][...] = (
            jnp.dot(p0, l0w[...], preferred_element_type=jnp.float32)
            + jnp.dot(p1, l1w[...], preferred_element_type=jnp.float32)
            + jnp.dot(p2, l2w[...], preferred_element_type=jnp.float32) + lb)
        dcol = dcol + jnp.dot(p2, dw[...][br * H:(br + 1) * H],
                              preferred_element_type=jnp.float32)
    dist[...] = dcol


def _f32(shape):
    return jax.ShapeDtypeStruct(shape, jnp.float32)


def kernel(x_1, edge_index_1, x_2, edge_index_2, batch_1, batch_2, params):
    p = params

    def prep(ei):
        src = jnp.concatenate([ei[0], jnp.zeros((PAD,), jnp.int32)])
        dst = jnp.concatenate([ei[1], jnp.full((PAD,), N, jnp.int32)])
        return (src.reshape(NCH, CHUNK), dst.reshape(NCH, CHUNK))

    s1, d1 = prep(edge_index_1)
    s2, d2 = prep(edge_index_2)
    srcq = jnp.concatenate([s1, s2], axis=0)
    dstq = jnp.concatenate([d1, d2], axis=0)
    zeros = jnp.zeros((NS * PT4,), jnp.float32)

    def to4(y):
        return (y.reshape(2, N_PAD, NS, FPT).transpose(0, 2, 1, 3)
                .reshape(NC * NS, PT4))

    def from4(a):
        return (a.reshape(2, NS, N_PAD, FPT).transpose(0, 2, 1, 3)
                .reshape(2, N_PAD, H))

    r = lambda a: a.reshape(1, -1)
    _segsum = _make_segsum()

    y0 = pl.pallas_call(_pre_body, out_shape=_f32((2, N_PAD, H)))(
        x_1, x_2, p['conv0_W1'])

    a0 = from4(_segsum(to4(y0), srcq, dstq, zeros))

    h1, y1 = pl.pallas_call(_post_body,
                            out_shape=(_f32((2, N, H)), _f32((2, N_PAD, H))))(
        y0, a0, r(p['conv0_b1']), r(p['conv0_bng']), r(p['conv0_bnb']),
        p['conv0_W2'], r(p['conv0_b2']), r(p['bn0_g']), r(p['bn0_b']),
        p['conv1_W1'])

    a1 = from4(_segsum(to4(y1), srcq, dstq, zeros))

    h2 = pl.pallas_call(_post_last_body, out_shape=_f32((2, N, H)))(
        y1, a1, r(p['conv1_b1']), r(p['conv1_bng']), r(p['conv1_bnb']),
        p['conv1_W2'], r(p['conv1_b2']), r(p['bn1_g']), r(p['bn1_b']))

    dist, o1, o2 = pl.pallas_call(
        _pool_body,
        out_shape=(_f32((G, 1)), _f32((G, C)), _f32((G, C))))(
        x_1, x_2, h1, h2, batch_1.reshape(N, 1), batch_2.reshape(N, 1),
        p['lin0_W'], p['lin1_W'], p['lin2_W'],
        r(p['lin0_b']), r(p['lin1_b']), r(p['lin2_b']),
        p['dis_W'], p['dis_b'].reshape(1, 1))

    return (dist, o1, o2)


# speedup vs baseline: 4.8202x; 4.8202x over previous
import functools

import jax
import jax.numpy as jnp
from jax import lax
from jax.experimental import pallas as pl
from jax.experimental.pallas import tpu as pltpu
from jax.experimental.pallas import tpu_sc as plsc

N = 10000
E = 320000
D = 128
H = 64
C = 128
G = 128

NC = 2
NS = 16
CHUNK = 128
STEPS = 160
HSTEPS = STEPS // 2
EPAD = STEPS * NS * CHUNK
PAD = EPAD - E
RPT = 632
N_PAD = NS * RPT


@functools.lru_cache(maxsize=1)
def _make_segsum():
    mesh = plsc.VectorSubcoreMesh(core_axis_name="c", subcore_axis_name="s",
                                  num_cores=NC, num_subcores=NS)

    @functools.partial(
        pl.kernel,
        out_type=jax.ShapeDtypeStruct((NC * N_PAD, H), jnp.float32),
        mesh=mesh,
        compiler_params=pltpu.CompilerParams(use_tc_tiling_on_sc=False),
        scratch_types=[
            pltpu.VMEM((HSTEPS, CHUNK), jnp.int32),
            pltpu.VMEM((HSTEPS, CHUNK), jnp.int32),
            pltpu.VMEM((2, CHUNK, H), jnp.float32),
            pltpu.VMEM_SHARED((N_PAD, H), jnp.float32),
            pltpu.VMEM_SHARED((N_PAD, H), jnp.float32),
            pltpu.SemaphoreType.DMA,
            pltpu.SemaphoreType.DMA,
        ],
    )
    def seg(y_hbm, srcs_hbm, dsts_hbm, z_hbm, out_hbm,
            src_v, dst_v, rows, y_sp, acc, sem, psem):
        c = lax.axis_index("c")
        s = lax.axis_index("s")
        w = c * NS + s
        pltpu.async_copy(y_hbm.at[pl.ds(c * N_PAD + s * RPT, RPT)],
                         y_sp.at[pl.ds(s * RPT, RPT)], psem)
        pltpu.async_copy(z_hbm.at[pl.ds(s * RPT, RPT)],
                         acc.at[pl.ds(s * RPT, RPT)], psem)
        pltpu.async_copy(srcs_hbm.at[2 * w], src_v, psem)
        pltpu.async_copy(dsts_hbm.at[2 * w], dst_v, psem)
        pltpu.make_async_copy(y_hbm.at[pl.ds(0, RPT)],
                              y_sp.at[pl.ds(0, RPT)], psem).wait()
        pltpu.make_async_copy(z_hbm.at[pl.ds(0, RPT)],
                              acc.at[pl.ds(0, RPT)], psem).wait()
        pltpu.make_async_copy(srcs_hbm.at[0], src_v, psem).wait()
        pltpu.make_async_copy(dsts_hbm.at[0], dst_v, psem).wait()
        plsc.subcore_barrier()

        def run_half():
            pltpu.async_copy(y_sp.at[src_v.at[0]], rows.at[0], sem)

            def body(g, carry):
                j0 = 2 * g
                j1 = j0 + 1
                pltpu.make_async_copy(y_sp.at[src_v.at[j0]],
                                      rows.at[0], sem).wait()
                pltpu.async_copy(y_sp.at[src_v.at[j1]], rows.at[1], sem)
                pltpu.sync_copy(rows.at[0], acc.at[dst_v.at[j0]], add=True)
                pltpu.make_async_copy(y_sp.at[src_v.at[j1]],
                                      rows.at[1], sem).wait()
                jn = lax.min(j1 + 1, HSTEPS - 1)
                pltpu.async_copy(y_sp.at[src_v.at[jn]], rows.at[0], sem)
                pltpu.sync_copy(rows.at[1], acc.at[dst_v.at[j1]], add=True)
                return carry

            lax.fori_loop(0, HSTEPS // 2, body, 0)
            pltpu.make_async_copy(y_sp.at[src_v.at[0]],
                                  rows.at[0], sem).wait()

        run_half()
        pltpu.sync_copy(srcs_hbm.at[2 * w + 1], src_v)
        pltpu.sync_copy(dsts_hbm.at[2 * w + 1], dst_v)
        run_half()
        plsc.subcore_barrier()
        pltpu.sync_copy(acc.at[pl.ds(s * RPT, RPT)],
                        out_hbm.at[pl.ds(c * N_PAD + s * RPT, RPT)])

    return seg


def _pre_body(x1, x2, w1, y):
    w = w1[...]
    z = jnp.zeros((N_PAD - N, H), jnp.float32)
    for br, x in enumerate((x1, x2)):
        y[br, :N] = jnp.dot(x[...], w, preferred_element_type=jnp.float32)
        y[br, N:] = z


def _bn(h, g, b):
    m = jnp.mean(h, axis=0, keepdims=True)
    v = jnp.mean((h - m) ** 2, axis=0, keepdims=True)
    return (h - m) * lax.rsqrt(v + 1e-5) * g + b


def _post_body(y, a, b1, bng, bnb, w2, b2, bg, bb, wn, h_out, yn_out):
    for br in range(2):
        h = y[br, :N] + a[br, :N] + b1[...]
        h = jnp.maximum(_bn(h, bng[...], bnb[...]), 0.0)
        h = jnp.dot(h, w2[...], preferred_element_type=jnp.float32) + b2[...]
        h = jnp.maximum(_bn(h, bg[...], bb[...]), 0.0)
        h_out[br] = h
        if yn_out is not None:
            yn_out[br, :N] = jnp.dot(h, wn[...],
                                     preferred_element_type=jnp.float32)
            yn_out[br, N:] = jnp.zeros((N_PAD - N, H), jnp.float32)


def _post_last_body(y, a, b1, bng, bnb, w2, b2, bg, bb, h_out):
    _post_body(y, a, b1, bng, bnb, w2, b2, bg, bb, None, h_out, None)


_CONTRACT0 = (((0,), (0,)), ((), ()))


def _pool_body(x1, x2, h1, h2, bt1, bt2, l0w, l1w, l2w,
               l0b, l1b, l2b, dw, db, dist, o1, o2):
    xs = (x1, x2)
    bts = (bt1, bt2)
    outs = (o1, o2)
    lb = l0b[...] + l1b[...] + l2b[...]
    dcol = db[...]
    for br in range(2):
        oh = (bts[br][...] == lax.broadcasted_iota(jnp.int32, (N, G), 1)
              ).astype(jnp.float32)
        p0 = lax.dot_general(oh, xs[br][...], _CONTRACT0,
                             preferred_element_type=jnp.float32)
        p1 = lax.dot_general(oh, h1[br], _CONTRACT0,
                             preferred_element_type=jnp.float32)
        p2 = lax.dot_general(oh, h2[br], _CONTRACT0,
                             preferred_element_type=jnp.float32)
        outs[br][...] = (
            jnp.dot(p0, l0w[...], preferred_element_type=jnp.float32)
            + jnp.dot(p1, l1w[...], preferred_element_type=jnp.float32)
            + jnp.dot(p2, l2w[...], preferred_element_type=jnp.float32) + lb)
        dcol = dcol + jnp.dot(p2, dw[...][br * H:(br + 1) * H],
                              preferred_element_type=jnp.float32)
    dist[...] = dcol


def _f32(shape):
    return jax.ShapeDtypeStruct(shape, jnp.float32)


def kernel(x_1, edge_index_1, x_2, edge_index_2, batch_1, batch_2, params):
    p = params

    def prep(ei):
        src = jnp.concatenate([ei[0], jnp.zeros((PAD,), jnp.int32)])
        dst = jnp.concatenate([ei[1], jnp.full((PAD,), N, jnp.int32)])
        return (src.reshape(NS * 2, HSTEPS, CHUNK),
                dst.reshape(NS * 2, HSTEPS, CHUNK))

    s1, d1 = prep(edge_index_1)
    s2, d2 = prep(edge_index_2)
    srcs = jnp.concatenate([s1, s2], axis=0)
    dsts = jnp.concatenate([d1, d2], axis=0)
    zeros = jnp.zeros((N_PAD, H), jnp.float32)

    r = lambda a: a.reshape(1, -1)
    _segsum = _make_segsum()

    y0 = pl.pallas_call(_pre_body, out_shape=_f32((2, N_PAD, H)))(
        x_1, x_2, p['conv0_W1'])

    a0 = _segsum(y0.reshape(NC * N_PAD, H), srcs, dsts,
                 zeros).reshape(2, N_PAD, H)

    h1, y1 = pl.pallas_call(_post_body,
                            out_shape=(_f32((2, N, H)), _f32((2, N_PAD, H))))(
        y0, a0, r(p['conv0_b1']), r(p['conv0_bng']), r(p['conv0_bnb']),
        p['conv0_W2'], r(p['conv0_b2']), r(p['bn0_g']), r(p['bn0_b']),
        p['conv1_W1'])

    a1 = _segsum(y1.reshape(NC * N_PAD, H), srcs, dsts,
                 zeros).reshape(2, N_PAD, H)

    h2 = pl.pallas_call(_post_last_body, out_shape=_f32((2, N, H)))(
        y1, a1, r(p['conv1_b1']), r(p['conv1_bng']), r(p['conv1_bnb']),
        p['conv1_W2'], r(p['conv1_b2']), r(p['bn1_g']), r(p['bn1_b']))

    dist, o1, o2 = pl.pallas_call(
        _pool_body,
        out_shape=(_f32((G, 1)), _f32((G, C)), _f32((G, C))))(
        x_1, x_2, h1, h2, batch_1.reshape(N, 1), batch_2.reshape(N, 1),
        p['lin0_W'], p['lin1_W'], p['lin2_W'],
        r(p['lin0_b']), r(p['lin1_b']), r(p['lin2_b']),
        p['dis_W'], p['dis_b'].reshape(1, 1))

    return (dist, o1, o2)


# speedup vs baseline: 4.8264x; 1.0013x over previous
import functools

import jax
import jax.numpy as jnp
from jax import lax
from jax.experimental import pallas as pl
from jax.experimental.pallas import tpu as pltpu
from jax.experimental.pallas import tpu_sc as plsc

N = 10000
E = 320000
D = 128
H = 64
C = 128
G = 128

NC = 2
NS = 16
CHUNK = 128
STEPS = 160
HSTEPS = STEPS // 2
EPAD = STEPS * NS * CHUNK
PAD = EPAD - E
RPT = 632
N_PAD = NS * RPT


@functools.lru_cache(maxsize=1)
def _make_segsum():
    mesh = plsc.VectorSubcoreMesh(core_axis_name="c", subcore_axis_name="s",
                                  num_cores=NC, num_subcores=NS)

    @functools.partial(
        pl.kernel,
        out_type=jax.ShapeDtypeStruct((NC * N_PAD, H), jnp.float32),
        mesh=mesh,
        compiler_params=pltpu.CompilerParams(use_tc_tiling_on_sc=False),
        scratch_types=[
            pltpu.VMEM((HSTEPS, CHUNK), jnp.int32),
            pltpu.VMEM((HSTEPS, CHUNK), jnp.int32),
            pltpu.VMEM((2, CHUNK, H), jnp.float32),
            pltpu.VMEM_SHARED((N_PAD, H), jnp.float32),
            pltpu.VMEM_SHARED((N_PAD, H), jnp.float32),
            pltpu.SemaphoreType.DMA,
            pltpu.SemaphoreType.DMA,
            pltpu.SemaphoreType.DMA,
        ],
    )
    def seg(y_hbm, srcs_hbm, dsts_hbm, z_hbm, out_hbm,
            src_v, dst_v, rows, y_sp, acc, sem, psem, ssem):
        c = lax.axis_index("c")
        s = lax.axis_index("s")
        w = c * NS + s
        pltpu.async_copy(y_hbm.at[pl.ds(c * N_PAD + s * RPT, RPT)],
                         y_sp.at[pl.ds(s * RPT, RPT)], psem)
        pltpu.async_copy(z_hbm.at[pl.ds(s * RPT, RPT)],
                         acc.at[pl.ds(s * RPT, RPT)], psem)
        pltpu.async_copy(srcs_hbm.at[2 * w], src_v, psem)
        pltpu.async_copy(dsts_hbm.at[2 * w], dst_v, psem)
        pltpu.make_async_copy(y_hbm.at[pl.ds(0, RPT)],
                              y_sp.at[pl.ds(0, RPT)], psem).wait()
        pltpu.make_async_copy(z_hbm.at[pl.ds(0, RPT)],
                              acc.at[pl.ds(0, RPT)], psem).wait()
        pltpu.make_async_copy(srcs_hbm.at[0], src_v, psem).wait()
        pltpu.make_async_copy(dsts_hbm.at[0], dst_v, psem).wait()
        plsc.subcore_barrier()

        def pair(j0, first):
            j1 = j0 + 1
            pltpu.make_async_copy(y_sp.at[src_v.at[j0]],
                                  rows.at[0], sem).wait()
            if not first:
                pltpu.make_async_copy(rows.at[1], acc.at[dst_v.at[0]],
                                      ssem).wait()
            pltpu.async_copy(y_sp.at[src_v.at[j1]], rows.at[1], sem)
            pltpu.sync_copy(rows.at[0], acc.at[dst_v.at[j0]], add=True)
            pltpu.make_async_copy(y_sp.at[src_v.at[j1]],
                                  rows.at[1], sem).wait()
            jn = lax.min(j1 + 1, HSTEPS - 1)
            pltpu.async_copy(y_sp.at[src_v.at[jn]], rows.at[0], sem)
            pltpu.async_copy(rows.at[1], acc.at[dst_v.at[j1]], ssem,
                             add=True)

        def run_half():
            pltpu.async_copy(y_sp.at[src_v.at[0]], rows.at[0], sem)
            pair(0, True)

            def body(g, carry):
                pair(2 * g, False)
                return carry

            lax.fori_loop(1, HSTEPS // 2, body, 0)
            pltpu.make_async_copy(y_sp.at[src_v.at[0]],
                                  rows.at[0], sem).wait()
            pltpu.make_async_copy(rows.at[1], acc.at[dst_v.at[0]],
                                  ssem).wait()

        run_half()
        pltpu.sync_copy(srcs_hbm.at[2 * w + 1], src_v)
        pltpu.sync_copy(dsts_hbm.at[2 * w + 1], dst_v)
        run_half()
        plsc.subcore_barrier()
        pltpu.sync_copy(acc.at[pl.ds(s * RPT, RPT)],
                        out_hbm.at[pl.ds(c * N_PAD + s * RPT, RPT)])

    return seg


def _pre_body(x1, x2, w1, y):
    w = w1[...]
    z = jnp.zeros((N_PAD - N, H), jnp.float32)
    for br, x in enumerate((x1, x2)):
        y[br, :N] = jnp.dot(x[...], w, preferred_element_type=jnp.float32)
        y[br, N:] = z


def _bn(h, g, b):
    m = jnp.mean(h, axis=0, keepdims=True)
    v = jnp.mean((h - m) ** 2, axis=0, keepdims=True)
    return (h - m) * lax.rsqrt(v + 1e-5) * g + b


def _post_body(y, a, b1, bng, bnb, w2, b2, bg, bb, wn, h_out, yn_out):
    for br in range(2):
        h = y[br, :N] + a[br, :N] + b1[...]
        h = jnp.maximum(_bn(h, bng[...], bnb[...]), 0.0)
        h = jnp.dot(h, w2[...], preferred_element_type=jnp.float32) + b2[...]
        h = jnp.maximum(_bn(h, bg[...], bb[...]), 0.0)
        h_out[br] = h
        if yn_out is not None:
            yn_out[br, :N] = jnp.dot(h, wn[...],
                                     preferred_element_type=jnp.float32)
            yn_out[br, N:] = jnp.zeros((N_PAD - N, H), jnp.float32)


def _post_last_body(y, a, b1, bng, bnb, w2, b2, bg, bb, h_out):
    _post_body(y, a, b1, bng, bnb, w2, b2, bg, bb, None, h_out, None)


_CONTRACT0 = (((0,), (0,)), ((), ()))


def _pool_body(x1, x2, h1, h2, bt1, bt2, l0w, l1w, l2w,
               l0b, l1b, l2b, dw, db, dist, o1, o2):
    xs = (x1, x2)
    bts = (bt1, bt2)
    outs = (o1, o2)
    lb = l0b[...] + l1b[...] + l2b[...]
    dcol = db[...]
    for br in range(2):
        oh = (bts[br][...] == lax.broadcasted_iota(jnp.int32, (N, G), 1)
              ).astype(jnp.float32)
        p0 = lax.dot_general(oh, xs[br][...], _CONTRACT0,
                             preferred_element_type=jnp.float32)
        p1 = lax.dot_general(oh, h1[br], _CONTRACT0,
                             preferred_element_type=jnp.float32)
        p2 = lax.dot_general(oh, h2[br], _CONTRACT0,
                             preferred_element_type=jnp.float32)
        outs[br][...] = (
            jnp.dot(p0, l0w[...], preferred_element_type=jnp.float32)
            + jnp.dot(p1, l1w[...], preferred_element_type=jnp.float32)
            + jnp.dot(p2, l2w[...], preferred_element_type=jnp.float32) + lb)
        dcol = dcol + jnp.dot(p2, dw[...][br * H:(br + 1) * H],
                              preferred_element_type=jnp.float32)
    dist[...] = dcol


def _f32(shape):
    return jax.ShapeDtypeStruct(shape, jnp.float32)


def kernel(x_1, edge_index_1, x_2, edge_index_2, batch_1, batch_2, params):
    p = params

    def prep(ei):
        src = jnp.concatenate([ei[0], jnp.zeros((PAD,), jnp.int32)])
        dst = jnp.concatenate([ei[1], jnp.full((PAD,), N, jnp.int32)])
        return (src.reshape(NS * 2, HSTEPS, CHUNK),
                dst.reshape(NS * 2, HSTEPS, CHUNK))

    s1, d1 = prep(edge_index_1)
    s2, d2 = prep(edge_index_2)
    srcs = jnp.concatenate([s1, s2], axis=0)
    dsts = jnp.concatenate([d1, d2], axis=0)
    zeros = jnp.zeros((N_PAD, H), jnp.float32)

    r = lambda a: a.reshape(1, -1)
    _segsum = _make_segsum()

    y0 = pl.pallas_call(_pre_body, out_shape=_f32((2, N_PAD, H)))(
        x_1, x_2, p['conv0_W1'])

    a0 = _segsum(y0.reshape(NC * N_PAD, H), srcs, dsts,
                 zeros).reshape(2, N_PAD, H)

    h1, y1 = pl.pallas_call(_post_body,
                            out_shape=(_f32((2, N, H)), _f32((2, N_PAD, H))))(
        y0, a0, r(p['conv0_b1']), r(p['conv0_bng']), r(p['conv0_bnb']),
        p['conv0_W2'], r(p['conv0_b2']), r(p['bn0_g']), r(p['bn0_b']),
        p['conv1_W1'])

    a1 = _segsum(y1.reshape(NC * N_PAD, H), srcs, dsts,
                 zeros).reshape(2, N_PAD, H)

    h2 = pl.pallas_call(_post_last_body, out_shape=_f32((2, N, H)))(
        y1, a1, r(p['conv1_b1']), r(p['conv1_bng']), r(p['conv1_bnb']),
        p['conv1_W2'], r(p['conv1_b2']), r(p['bn1_g']), r(p['bn1_b']))

    dist, o1, o2 = pl.pallas_call(
        _pool_body,
        out_shape=(_f32((G, 1)), _f32((G, C)), _f32((G, C))))(
        x_1, x_2, h1, h2, batch_1.reshape(N, 1), batch_2.reshape(N, 1),
        p['lin0_W'], p['lin1_W'], p['lin2_W'],
        r(p['lin0_b']), r(p['lin1_b']), r(p['lin2_b']),
        p['dis_W'], p['dis_b'].reshape(1, 1))

    return (dist, o1, o2)
